# Initial kernel scaffold; baseline (speedup 1.0000x reference)
import jax
import jax.numpy as jnp
from jax.experimental import pallas as pl


def kernel(feat, edge_index, W):
    raise NotImplementedError("write your pallas kernel here")


# speedup vs baseline: 6.6350x; 6.6350x over previous
import functools

import jax
import jax.numpy as jnp
from jax import lax
from jax.experimental import pallas as pl
from jax.experimental.pallas import tpu as pltpu
from jax.experimental.pallas import tpu_sc as plsc

N = 10000
E = 320000
D = 128

NC = 2
NS = 16
NW = NC * NS
DE = 144
N_PAD = 10240
K = 80
EPW = E // NW
CPW = EPW // K
ROWS_PER_TILE = N_PAD // NS


def _sc_segment_sum(feat_ext, src2d, dst2d, zeros_blk):
  mesh = plsc.VectorSubcoreMesh(
      core_axis_name="c", subcore_axis_name="s",
      num_cores=NC, num_subcores=NS)

  @functools.partial(
      pl.kernel,
      out_type=jax.ShapeDtypeStruct((NC * N_PAD, DE), jnp.float32),
      mesh=mesh,
      scratch_types=[
          pltpu.MemorySpace.VMEM_SHARED((N_PAD, DE), jnp.float32),
          pltpu.VMEM((CPW, K), jnp.int32),
          pltpu.VMEM((CPW, K), jnp.int32),
          pltpu.VMEM((K, DE), jnp.float32),
          pltpu.SemaphoreType.DMA,
      ],
      compiler_params=pltpu.CompilerParams(use_tc_tiling_on_sc=False),
  )
  def k(feat_hbm, src_hbm, dst_hbm, zeros_hbm, hp_hbm,
        h_sh, idx_src, idx_dst, rows, sem):
    c = lax.axis_index("c")
    s = lax.axis_index("s")
    wid = c * NS + s

    pltpu.sync_copy(zeros_hbm, rows)
    rbase = s * ROWS_PER_TILE
    for z in range(ROWS_PER_TILE // K):
      pltpu.sync_copy(rows, h_sh.at[pl.ds(rbase + z * K, K)])
    plsc.subcore_barrier()

    pltpu.sync_copy(src_hbm.at[wid], idx_src)
    pltpu.sync_copy(dst_hbm.at[wid], idx_dst)

    def body(j, carry):
      pltpu.async_copy(feat_hbm.at[idx_src.at[j]], rows, sem).wait()
      pltpu.sync_copy(rows, h_sh.at[idx_dst.at[j]], add=True)
      return carry

    lax.fori_loop(0, CPW, body, 0, unroll=False)
    plsc.subcore_barrier()

    pltpu.sync_copy(h_sh.at[pl.ds(rbase, ROWS_PER_TILE)],
                    hp_hbm.at[pl.ds(c * N_PAD + rbase, ROWS_PER_TILE)])

  return k(feat_ext, src2d, dst2d, zeros_blk)


def _tc_body(hp_ref, w_ref, o_ref):
  hp = hp_ref[...]
  h = hp[0] + hp[1]
  y = jnp.dot(h[:, :D], w_ref[...], preferred_element_type=jnp.float32)
  deg = h[:, D:D + 1]
  norm = lax.rsqrt(jnp.maximum(deg, 1.0))
  o_ref[...] = y * norm


def _tc_matmul_norm(hp, W):
  BM = 1024
  grid = (N_PAD // BM,)
  hp3 = hp.reshape(NC, N_PAD, DE)
  return pl.pallas_call(
      _tc_body,
      grid=grid,
      in_specs=[
          pl.BlockSpec((NC, BM, DE), lambda i: (0, i, 0)),
          pl.BlockSpec((D, D), lambda i: (0, 0)),
      ],
      out_specs=pl.BlockSpec((BM, D), lambda i: (i, 0)),
      out_shape=jax.ShapeDtypeStruct((N_PAD, D), jnp.float32),
  )(hp3, W)


def kernel(feat, edge_index, W):
  feat_ext = jnp.pad(feat, ((0, 0), (0, DE - D)))
  feat_ext = feat_ext.at[:, D].set(1.0)
  src2d = edge_index[0].reshape(NW, CPW, K)
  dst2d = edge_index[1].reshape(NW, CPW, K)
  zeros_blk = jnp.zeros((K, DE), jnp.float32)
  hp = _sc_segment_sum(feat_ext, src2d, dst2d, zeros_blk)
  rst = _tc_matmul_norm(hp, W)
  return rst[:N]


# speedup vs baseline: 9.8126x; 1.4789x over previous
import functools

import jax
import jax.numpy as jnp
from jax import lax
from jax.experimental import pallas as pl
from jax.experimental.pallas import tpu as pltpu
from jax.experimental.pallas import tpu_sc as plsc

N = 10000
E = 320000
D = 128

NC = 2
NS = 16
NW = NC * NS
DE = 144
N_PAD = 10240
K = 80
EPW = E // NW
CPW = EPW // K
CPR = 25
NREFILL = CPW // CPR
ROWS_PER_TILE = N_PAD // NS


def _sc_segment_sum(feat_ext, src3d, dst3d, zeros_blk):
  mesh = plsc.VectorSubcoreMesh(
      core_axis_name="c", subcore_axis_name="s",
      num_cores=NC, num_subcores=NS)

  @functools.partial(
      pl.kernel,
      out_type=jax.ShapeDtypeStruct((NC, N_PAD, DE), jnp.float32),
      mesh=mesh,
      scratch_types=[
          pltpu.MemorySpace.VMEM_SHARED((N_PAD, DE), jnp.float32),
          pltpu.VMEM((CPR, K), jnp.int32),
          pltpu.VMEM((CPR, K), jnp.int32),
          pltpu.VMEM((K, DE), jnp.float32),
          pltpu.VMEM((K, DE), jnp.float32),
          pltpu.SemaphoreType.DMA,
          pltpu.SemaphoreType.DMA,
      ],
      compiler_params=pltpu.CompilerParams(use_tc_tiling_on_sc=False),
  )
  def k(feat_hbm, src_hbm, dst_hbm, zeros_hbm, hp_hbm,
        h_sh, idx_src, idx_dst, rows_a, rows_b, sem_a, sem_b):
    c = lax.axis_index("c")
    s = lax.axis_index("s")
    wid = c * NS + s

    pltpu.sync_copy(zeros_hbm, rows_a)
    rbase = s * ROWS_PER_TILE
    for z in range(ROWS_PER_TILE // K):
      pltpu.sync_copy(rows_a, h_sh.at[pl.ds(rbase + z * K, K)])
    plsc.subcore_barrier()

    def refill(m, carry):
      pltpu.sync_copy(src_hbm.at[wid, pl.ds(m * CPR, CPR)], idx_src)
      pltpu.sync_copy(dst_hbm.at[wid, pl.ds(m * CPR, CPR)], idx_dst)

      pltpu.async_copy(feat_hbm.at[idx_src.at[0]], rows_a, sem_a)

      def step(j, carry2):
        ja = 2 * j
        pltpu.async_copy(feat_hbm.at[idx_src.at[ja + 1]], rows_b, sem_b)
        pltpu.make_async_copy(feat_hbm.at[idx_src.at[ja]], rows_a, sem_a).wait()
        pltpu.sync_copy(rows_a, h_sh.at[idx_dst.at[ja]], add=True)
        pltpu.async_copy(feat_hbm.at[idx_src.at[ja + 2]], rows_a, sem_a)
        pltpu.make_async_copy(
            feat_hbm.at[idx_src.at[ja + 1]], rows_b, sem_b).wait()
        pltpu.sync_copy(rows_b, h_sh.at[idx_dst.at[ja + 1]], add=True)
        return carry2

      lax.fori_loop(0, (CPR - 1) // 2, step, 0, unroll=False)
      pltpu.make_async_copy(
          feat_hbm.at[idx_src.at[CPR - 1]], rows_a, sem_a).wait()
      pltpu.sync_copy(rows_a, h_sh.at[idx_dst.at[CPR - 1]], add=True)
      return carry

    lax.fori_loop(0, NREFILL, refill, 0, unroll=False)
    plsc.subcore_barrier()

    pltpu.sync_copy(h_sh.at[pl.ds(rbase, ROWS_PER_TILE)],
                    hp_hbm.at[c, pl.ds(rbase, ROWS_PER_TILE)])

  return k(feat_ext, src3d, dst3d, zeros_blk)


def _tc_body(hp_ref, w_ref, o_ref):
  hp = hp_ref[...]
  h = hp[0] + hp[1]
  y = jnp.dot(h[:, :D], w_ref[...], preferred_element_type=jnp.float32)
  deg = h[:, D:D + 1]
  norm = lax.rsqrt(jnp.maximum(deg, 1.0))
  o_ref[...] = y * norm


def _tc_matmul_norm(hp, W):
  BM = 1000
  grid = (N // BM,)
  return pl.pallas_call(
      _tc_body,
      grid=grid,
      in_specs=[
          pl.BlockSpec((NC, BM, DE), lambda i: (0, i, 0)),
          pl.BlockSpec((D, D), lambda i: (0, 0)),
      ],
      out_specs=pl.BlockSpec((BM, D), lambda i: (i, 0)),
      out_shape=jax.ShapeDtypeStruct((N, D), jnp.float32),
  )(hp, W)


def kernel(feat, edge_index, W):
  ones_col = jnp.ones((N, 1), jnp.float32)
  pad_cols = jnp.zeros((N, DE - D - 1), jnp.float32)
  feat_ext = jnp.concatenate([feat, ones_col, pad_cols], axis=1)
  src3d = edge_index[0].reshape(NW, CPW, K)
  dst3d = edge_index[1].reshape(NW, CPW, K)
  zeros_blk = jnp.zeros((K, DE), jnp.float32)
  hp = _sc_segment_sum(feat_ext, src3d, dst3d, zeros_blk)
  return _tc_matmul_norm(hp, W)


# speedup vs baseline: 11.8471x; 1.2073x over previous
import functools

import jax
import jax.numpy as jnp
from jax import lax
from jax.experimental import pallas as pl
from jax.experimental.pallas import tpu as pltpu
from jax.experimental.pallas import tpu_sc as plsc

N = 10000
E = 320000
D = 128

NC = 2
NS = 16
NW = NC * NS
N_PAD = 10240
K = 80
EPW = E // NW
CPW = EPW // K
CPR = 25
NREFILL = CPW // CPR
ROWS_PER_TILE = N_PAD // NS
L = 16


def _sc_segment_sum(feat, src3d, dst3d, zeros_blk):
  mesh = plsc.VectorSubcoreMesh(
      core_axis_name="c", subcore_axis_name="s",
      num_cores=NC, num_subcores=NS)

  @functools.partial(
      pl.kernel,
      out_type=(jax.ShapeDtypeStruct((NC, N_PAD, D), jnp.float32),
                jax.ShapeDtypeStruct((NW, N_PAD), jnp.float32)),
      mesh=mesh,
      scratch_types=[
          pltpu.MemorySpace.VMEM_SHARED((N_PAD, D), jnp.float32),
          pltpu.VMEM((CPR, K), jnp.int32),
          pltpu.VMEM((CPR, K), jnp.int32),
          pltpu.VMEM((K, D), jnp.float32),
          pltpu.VMEM((K, D), jnp.float32),
          pltpu.VMEM((N_PAD,), jnp.float32),
          pltpu.SemaphoreType.DMA,
          pltpu.SemaphoreType.DMA,
      ],
      compiler_params=pltpu.CompilerParams(
          use_tc_tiling_on_sc=False, needs_layout_passes=False),
  )
  def k(feat_hbm, src_hbm, dst_hbm, zeros_hbm, hp_hbm, dp_hbm,
        h_sh, idx_src, idx_dst, rows_a, rows_b, deg, sem_a, sem_b):
    c = lax.axis_index("c")
    s = lax.axis_index("s")
    wid = c * NS + s

    zvec = jnp.zeros((L,), jnp.float32)

    def zero_deg(i, carry):
      deg[pl.ds(i * L, L)] = zvec
      return carry

    lax.fori_loop(0, N_PAD // L, zero_deg, 0, unroll=False)
    pltpu.sync_copy(zeros_hbm, rows_a)
    rbase = s * ROWS_PER_TILE
    for z in range(ROWS_PER_TILE // K):
      pltpu.sync_copy(rows_a, h_sh.at[pl.ds(rbase + z * K, K)])
    plsc.subcore_barrier()

    ones_v = jnp.ones((L,), jnp.float32)

    def deg_accum(j):
      def dstep(i, carry):
        idx = idx_dst[j, pl.ds(i * L, L)]
        plsc.addupdate_scatter(deg, [idx], ones_v)
        return carry

      lax.fori_loop(0, K // L, dstep, 0, unroll=False)

    def refill(m, carry):
      pltpu.sync_copy(src_hbm.at[wid, pl.ds(m * CPR, CPR)], idx_src)
      pltpu.sync_copy(dst_hbm.at[wid, pl.ds(m * CPR, CPR)], idx_dst)

      pltpu.async_copy(feat_hbm.at[idx_src.at[0]], rows_a, sem_a)

      def step(j, carry2):
        ja = 2 * j
        pltpu.async_copy(feat_hbm.at[idx_src.at[ja + 1]], rows_b, sem_b)
        deg_accum(ja)
        pltpu.make_async_copy(feat_hbm.at[idx_src.at[ja]], rows_a, sem_a).wait()
        pltpu.sync_copy(rows_a, h_sh.at[idx_dst.at[ja]], add=True)
        pltpu.async_copy(feat_hbm.at[idx_src.at[ja + 2]], rows_a, sem_a)
        deg_accum(ja + 1)
        pltpu.make_async_copy(
            feat_hbm.at[idx_src.at[ja + 1]], rows_b, sem_b).wait()
        pltpu.sync_copy(rows_b, h_sh.at[idx_dst.at[ja + 1]], add=True)
        return carry2

      lax.fori_loop(0, (CPR - 1) // 2, step, 0, unroll=False)
      deg_accum(CPR - 1)
      pltpu.make_async_copy(
          feat_hbm.at[idx_src.at[CPR - 1]], rows_a, sem_a).wait()
      pltpu.sync_copy(rows_a, h_sh.at[idx_dst.at[CPR - 1]], add=True)
      return carry

    lax.fori_loop(0, NREFILL, refill, 0, unroll=False)
    pltpu.sync_copy(deg, dp_hbm.at[wid])
    plsc.subcore_barrier()

    pltpu.sync_copy(h_sh.at[pl.ds(rbase, ROWS_PER_TILE)],
                    hp_hbm.at[c, pl.ds(rbase, ROWS_PER_TILE)])

  return k(feat, src3d, dst3d, zeros_blk)


def _tc_body(hp_ref, dp_ref, w_ref, o_ref):
  hp = hp_ref[...]
  h = hp[0] + hp[1]
  y = jnp.dot(h, w_ref[...], preferred_element_type=jnp.float32)
  deg = jnp.sum(dp_ref[...].T, axis=1, keepdims=True)
  norm = lax.rsqrt(jnp.maximum(deg, 1.0))
  o_ref[...] = y * norm


def _tc_matmul_norm(hp, dp, W):
  BM = 1024
  grid = (N_PAD // BM,)
  return pl.pallas_call(
      _tc_body,
      grid=grid,
      in_specs=[
          pl.BlockSpec((NC, BM, D), lambda i: (0, i, 0)),
          pl.BlockSpec((NW, BM), lambda i: (0, i)),
          pl.BlockSpec((D, D), lambda i: (0, 0)),
      ],
      out_specs=pl.BlockSpec((BM, D), lambda i: (i, 0)),
      out_shape=jax.ShapeDtypeStruct((N_PAD, D), jnp.float32),
  )(hp, dp, W)


def kernel(feat, edge_index, W):
  src3d = edge_index[0].reshape(NW, CPW, K)
  dst3d = edge_index[1].reshape(NW, CPW, K)
  zeros_blk = jnp.zeros((K, D), jnp.float32)
  hp, dp = _sc_segment_sum(feat, src3d, dst3d, zeros_blk)
  return _tc_matmul_norm(hp, dp, W)[:N]


# speedup vs baseline: 12.4755x; 1.0530x over previous
import functools

import jax
import jax.numpy as jnp
from jax import lax
from jax.experimental import pallas as pl
from jax.experimental.pallas import tpu as pltpu
from jax.experimental.pallas import tpu_sc as plsc

N = 10000
E = 320000
D = 128

NC = 2
NS = 16
NW = NC * NS
N_PAD = 10240
K = 80
EPW = E // NW
CPW = EPW // K
ROWS_PER_TILE = N_PAD // NS
L = 16


def _sc_segment_sum(feat, edge4d, zeros_blk):
  mesh = plsc.VectorSubcoreMesh(
      core_axis_name="c", subcore_axis_name="s",
      num_cores=NC, num_subcores=NS)

  @functools.partial(
      pl.kernel,
      out_type=(jax.ShapeDtypeStruct((2 * NC, N_PAD, D), jnp.bfloat16),
                jax.ShapeDtypeStruct((NW, N_PAD), jnp.float32)),
      mesh=mesh,
      scratch_types=[
          pltpu.MemorySpace.VMEM_SHARED((N_PAD, D), jnp.bfloat16),
          pltpu.MemorySpace.VMEM_SHARED((N_PAD, D), jnp.bfloat16),
          pltpu.VMEM((CPW, K), jnp.int32),
          pltpu.VMEM((CPW, K), jnp.int32),
          pltpu.VMEM((3, K, D), jnp.bfloat16),
          pltpu.VMEM((N_PAD,), jnp.float32),
          pltpu.SemaphoreType.DMA,
          pltpu.SemaphoreType.DMA,
          pltpu.SemaphoreType.DMA,
          pltpu.SemaphoreType.DMA,
          pltpu.SemaphoreType.DMA,
          pltpu.SemaphoreType.DMA,
      ],
      compiler_params=pltpu.CompilerParams(
          use_tc_tiling_on_sc=False, needs_layout_passes=False),
  )
  def k(feat_hbm, edge_hbm, zeros_hbm, hp_hbm, dp_hbm,
        h_sha, h_shb, idx_src, idx_dst, rows, deg,
        gs0, gs1, gs2, ss0, ss1, ss2):
    gsem = (gs0, gs1, gs2)
    ssem = (ss0, ss1, ss2)
    accs = (h_sha, h_shb)
    c = lax.axis_index("c")
    s = lax.axis_index("s")
    wid = c * NS + s

    zvec = jnp.zeros((L,), jnp.float32)

    def zero_deg(i, carry):
      deg[pl.ds(i * L, L)] = zvec
      return carry

    lax.fori_loop(0, N_PAD // L, zero_deg, 0, unroll=False)
    pltpu.sync_copy(zeros_hbm, rows.at[0])
    rbase = s * ROWS_PER_TILE
    for acc in accs:
      for z in range(ROWS_PER_TILE // K):
        pltpu.sync_copy(rows.at[0], acc.at[pl.ds(rbase + z * K, K)])
    plsc.subcore_barrier()

    ones_v = jnp.ones((L,), jnp.float32)

    def deg_accum(j):
      def dstep(i, carry):
        idx = idx_dst[j, pl.ds(i * L, L)]
        plsc.addupdate_scatter(deg, [idx], ones_v)
        return carry

      lax.fori_loop(0, K // L, dstep, 0, unroll=False)

    def gather(t, b):
      pltpu.async_copy(feat_hbm.at[idx_src.at[t]], rows.at[b], gsem[b])

    def wait_gather(t, b):
      pltpu.make_async_copy(
          feat_hbm.at[idx_src.at[t]], rows.at[b], gsem[b]).wait()

    def scatter(t, b, a):
      pltpu.async_copy(
          rows.at[b], accs[a].at[idx_dst.at[t]], ssem[b], add=True)

    def wait_scatter(t, b, a):
      pltpu.make_async_copy(
          rows.at[b], accs[a].at[idx_dst.at[t]], ssem[b]).wait()

    ci = pltpu.async_copy(edge_hbm.at[0, wid], idx_src, gs0)
    cj = pltpu.async_copy(edge_hbm.at[1, wid], idx_dst, gs1)
    ci.wait()
    cj.wait()

    NMAIN = (CPW - 5) // 6
    for b in range(3):
      gather(b, b)

    def step(j, carry2):
      t0 = 6 * j
      for half in range(2):
        for b in range(3):
          t = t0 + 3 * half + b
          wait_gather(t, b)
          scatter(t, b, (3 * half + b) % 2)
          deg_accum(t)
        for b in range(3):
          t = t0 + 3 * half + b
          wait_scatter(t, b, (3 * half + b) % 2)
          gather(t + 3, b)
      return carry2

    lax.fori_loop(0, NMAIN, step, 0, unroll=False)
    t1 = 6 * NMAIN
    for t in range(t1, t1 + 3):
      wait_gather(t, t % 3)
      scatter(t, t % 3, t % 2)
      deg_accum(t)
    for t in range(t1 + 3, CPW):
      b = t % 3
      wait_scatter(t - 3, b, (t - 3) % 2)
      gather(t, b)
      wait_gather(t, b)
      scatter(t, b, t % 2)
      deg_accum(t)
    for t in range(CPW - 3, CPW):
      wait_scatter(t, t % 3, t % 2)
    pltpu.sync_copy(deg, dp_hbm.at[wid])
    plsc.subcore_barrier()

    for a in range(2):
      pltpu.sync_copy(accs[a].at[pl.ds(rbase, ROWS_PER_TILE)],
                      hp_hbm.at[2 * c + a, pl.ds(rbase, ROWS_PER_TILE)])

  return k(feat, edge4d, zeros_blk)


def _tc_body(hp_ref, dp_ref, w_ref, o_ref):
  hp = hp_ref[...].astype(jnp.float32)
  h = (hp[0] + hp[1]) + (hp[2] + hp[3])
  y = jnp.dot(h, w_ref[...], preferred_element_type=jnp.float32)
  deg = jnp.sum(dp_ref[...].T, axis=1, keepdims=True)
  norm = lax.rsqrt(jnp.maximum(deg, 1.0))
  o_ref[...] = y * norm


def _tc_matmul_norm(hp, dp, W):
  BM = 1024
  grid = (N_PAD // BM,)
  return pl.pallas_call(
      _tc_body,
      grid=grid,
      in_specs=[
          pl.BlockSpec((2 * NC, BM, D), lambda i: (0, i, 0)),
          pl.BlockSpec((NW, BM), lambda i: (0, i)),
          pl.BlockSpec((D, D), lambda i: (0, 0)),
      ],
      out_specs=pl.BlockSpec((BM, D), lambda i: (i, 0)),
      out_shape=jax.ShapeDtypeStruct((N, D), jnp.float32),
  )(hp, dp, W)


def kernel(feat, edge_index, W):
  edge4d = edge_index.reshape(2, NW, CPW, K)
  zeros_blk = jnp.zeros((K, D), jnp.bfloat16)
  hp, dp = _sc_segment_sum(feat.astype(jnp.bfloat16), edge4d, zeros_blk)
  return _tc_matmul_norm(hp, dp, W)
